# baseline (speedup 1.0000x reference)
import jax
import jax.numpy as jnp
from jax import lax
from jax.experimental import pallas as pl
from jax.experimental.pallas import tpu as pltpu
from jax.experimental.pallas import tpu_sc as plsc

_LANES = 16
_NUM_WORKERS = 32


def _make_kernel(H, W, F):
    assert H == _NUM_WORKERS
    assert W % _LANES == 0 and F % _LANES == 0

    mesh = plsc.VectorSubcoreMesh(
        core_axis_name="c", subcore_axis_name="s", num_cores=2, num_subcores=16
    )

    def body(row_hbm, col_hbm, out_hbm, col_v, rb_v, row_v, sem_c, sem_r, sem_out):
        cid = lax.axis_index("c")
        sid = lax.axis_index("s")
        i = sid * 2 + cid

        w_col = pltpu.async_copy(
            col_hbm, out_hbm.at[0, i, :, pl.ds(0, F)], sem_c
        )
        c_row = pltpu.async_copy(row_hbm.at[i], row_v, sem_r)
        c_row.wait()
        for v in range(F // _LANES):
            reg = row_v[pl.ds(v * _LANES, _LANES)]
            for j in range(W):
                rb_v[j, pl.ds(v * _LANES, _LANES)] = reg
        w_row = pltpu.async_copy(rb_v, out_hbm.at[0, i, :, pl.ds(F, F)], sem_out)
        w_row.wait()
        w_col.wait()

    return pl.kernel(
        body,
        out_type=jax.ShapeDtypeStruct((1, H, W, 2 * F), jnp.float32),
        mesh=mesh,
        scratch_types=[
            pltpu.VMEM((W, F), jnp.float32),
            pltpu.VMEM((W, F), jnp.float32),
            pltpu.VMEM((F,), jnp.float32),
            pltpu.SemaphoreType.DMA,
            pltpu.SemaphoreType.DMA,
            pltpu.SemaphoreType.DMA,
        ],
    )


def kernel(inputs, row_embed, col_embed):
    H = inputs.shape[1]
    W = inputs.shape[2]
    F = row_embed.shape[-1]
    return _make_kernel(H, W, F)(row_embed, col_embed)


# speedup vs baseline: 2.2084x; 2.2084x over previous
import jax
import jax.numpy as jnp
from jax import lax
from jax.experimental import pallas as pl
from jax.experimental.pallas import tpu as pltpu
from jax.experimental.pallas import tpu_sc as plsc

_LANES = 16
_NUM_WORKERS = 32


def _make_kernel(H, W, F):
    assert H == _NUM_WORKERS
    assert W % _LANES == 0 and F % _LANES == 0

    mesh = plsc.VectorSubcoreMesh(
        core_axis_name="c", subcore_axis_name="s", num_cores=2, num_subcores=16
    )

    def body(row_hbm, col_hbm, out_hbm, col_v, rb_v, row_v, sem_c, sem_r, sem_out):
        cid = lax.axis_index("c")
        sid = lax.axis_index("s")
        i = sid * 2 + cid

        c_col = pltpu.async_copy(col_hbm, col_v, sem_c)
        c_row = pltpu.async_copy(row_hbm.at[i], row_v, sem_r)
        c_row.wait()
        regs = [row_v[pl.ds(v * _LANES, _LANES)] for v in range(F // _LANES)]

        def splat_row(j, _):
            for v in range(F // _LANES):
                rb_v[j, pl.ds(v * _LANES, _LANES)] = regs[v]
            return _

        lax.fori_loop(0, W, splat_row, 0)
        c_col.wait()
        w_col = pltpu.async_copy(col_v, out_hbm.at[0, i, :, pl.ds(0, F)], sem_out)
        w_row = pltpu.async_copy(rb_v, out_hbm.at[0, i, :, pl.ds(F, F)], sem_out)
        w_col.wait()
        w_row.wait()

    return pl.kernel(
        body,
        out_type=jax.ShapeDtypeStruct((1, H, W, 2 * F), jnp.float32),
        mesh=mesh,
        scratch_types=[
            pltpu.VMEM((W, F), jnp.float32),
            pltpu.VMEM((W, F), jnp.float32),
            pltpu.VMEM((F,), jnp.float32),
            pltpu.SemaphoreType.DMA,
            pltpu.SemaphoreType.DMA,
            pltpu.SemaphoreType.DMA,
        ],
    )


def kernel(inputs, row_embed, col_embed):
    H = inputs.shape[1]
    W = inputs.shape[2]
    F = row_embed.shape[-1]
    return _make_kernel(H, W, F)(row_embed, col_embed)


# speedup vs baseline: 2.4486x; 1.1088x over previous
import jax
import jax.numpy as jnp
from jax import lax
from jax.experimental import pallas as pl
from jax.experimental.pallas import tpu as pltpu
from jax.experimental.pallas import tpu_sc as plsc

_LANES = 16
_NUM_WORKERS = 32


def _make_kernel(H, W, F):
    assert H == _NUM_WORKERS and W == H
    assert F % _LANES == 0

    mesh = plsc.VectorSubcoreMesh(
        core_axis_name="c", subcore_axis_name="s", num_cores=2, num_subcores=16
    )

    def body(
        row_hbm, col_hbm, out_hbm, col_v, row_v, cb_v, rb_v, sem_c, sem_r, sem_out
    ):
        cid = lax.axis_index("c")
        sid = lax.axis_index("s")
        i = sid * 2 + cid

        c_col = pltpu.async_copy(col_hbm.at[i], col_v, sem_c)
        c_row = pltpu.async_copy(row_hbm.at[i], row_v, sem_r)
        c_col.wait()
        c_row.wait()
        cregs = [col_v[pl.ds(v * _LANES, _LANES)] for v in range(F // _LANES)]
        rregs = [row_v[pl.ds(v * _LANES, _LANES)] for v in range(F // _LANES)]

        def splat(j, _):
            for v in range(F // _LANES):
                cb_v[j, pl.ds(v * _LANES, _LANES)] = cregs[v]
                rb_v[j, pl.ds(v * _LANES, _LANES)] = rregs[v]
            return _

        lax.fori_loop(0, H, splat, 0)
        w_col = pltpu.async_copy(cb_v, out_hbm.at[0, :, i, pl.ds(0, F)], sem_out)
        w_row = pltpu.async_copy(rb_v, out_hbm.at[0, i, :, pl.ds(F, F)], sem_out)
        w_col.wait()
        w_row.wait()

    return pl.kernel(
        body,
        out_type=jax.ShapeDtypeStruct((1, H, W, 2 * F), jnp.float32),
        mesh=mesh,
        scratch_types=[
            pltpu.VMEM((F,), jnp.float32),
            pltpu.VMEM((F,), jnp.float32),
            pltpu.VMEM((H, F), jnp.float32),
            pltpu.VMEM((W, F), jnp.float32),
            pltpu.SemaphoreType.DMA,
            pltpu.SemaphoreType.DMA,
            pltpu.SemaphoreType.DMA,
        ],
    )


def kernel(inputs, row_embed, col_embed):
    H = inputs.shape[1]
    W = inputs.shape[2]
    F = row_embed.shape[-1]
    return _make_kernel(H, W, F)(row_embed, col_embed)
